# Initial kernel scaffold; baseline (speedup 1.0000x reference)
import jax
import jax.numpy as jnp
from jax.experimental import pallas as pl


def kernel(x, edge_index, target_n, target_g, node_idx, W, Wb, b, att_src, att_dst, w1, b1, w2, b2, w3, b3):
    raise NotImplementedError("write your pallas kernel here")


# speedup vs baseline: 16.3823x; 16.3823x over previous
import functools

import jax
import jax.numpy as jnp
from jax import lax
from jax.experimental import pallas as pl
from jax.experimental.pallas import tpu as pltpu
from jax.experimental.pallas import tpu_sc as plsc

NC = 2
NS = 16
NW = NC * NS


def _leaky(v, slope):
    return jnp.where(v >= 0, v, v * slope)


def _tc_stats_body(p1a_ref, p1b_ref, p2a_ref, p2b_ref, x_ref, nidx_ref,
                   W_ref, Wb_ref, b_ref, att_s_ref, att_d_ref,
                   h1_ref, h2_ref, asrc_ref, adst_ref):
    state = jnp.concatenate(
        [p1a_ref[...] + p1b_ref[...], p2a_ref[...] + p2b_ref[...],
         x_ref[...]], axis=1)
    h = state @ jnp.concatenate([W_ref[...], Wb_ref[...]], axis=0) + b_ref[...]
    m = att_s_ref.shape[0]
    n_blk = state.shape[0]
    onehot = (nidx_ref[...] == lax.broadcasted_iota(jnp.int32, (n_blk, m), 1)
              ).astype(jnp.float32)
    a_s = onehot @ att_s_ref[...]
    a_d = onehot @ att_d_ref[...]
    hh = h.shape[1] // 2
    h1_ref[...] = h[:, :hh]
    h2_ref[...] = h[:, hh:]
    asrc_ref[...] = jnp.sum(h * a_s, axis=1, keepdims=True)
    adst_ref[...] = jnp.sum(h * a_d, axis=1, keepdims=True)


def _tc_stats(p1, p2, xp, nidxp, W, Wb, b2d, att_src, att_dst, n_pad):
    h_dim = W.shape[0]
    hh = h_dim // 2
    grid = 4
    blk = n_pad // grid
    nb2 = n_pad // blk
    full = lambda shape: pl.BlockSpec(shape, lambda g: (0, 0))
    return pl.pallas_call(
        _tc_stats_body,
        grid=(grid,),
        in_specs=[
            pl.BlockSpec((blk, hh), lambda g: (g, 0)),
            pl.BlockSpec((blk, hh), lambda g, nb2=nb2: (g + nb2, 0)),
            pl.BlockSpec((blk, hh), lambda g: (g, 0)),
            pl.BlockSpec((blk, hh), lambda g, nb2=nb2: (g + nb2, 0)),
            pl.BlockSpec((blk, 2), lambda g: (g, 0)),
            pl.BlockSpec((blk, 1), lambda g: (g, 0)),
            full((h_dim, h_dim)),
            full((2, h_dim)),
            full((1, h_dim)),
            full((att_src.shape[0], h_dim)),
            full((att_src.shape[0], h_dim)),
        ],
        out_specs=[
            pl.BlockSpec((blk, hh), lambda g: (g, 0)),
            pl.BlockSpec((blk, hh), lambda g: (g, 0)),
            pl.BlockSpec((blk, 1), lambda g: (g, 0)),
            pl.BlockSpec((blk, 1), lambda g: (g, 0)),
        ],
        out_shape=[
            jax.ShapeDtypeStruct((n_pad, hh), jnp.float32),
            jax.ShapeDtypeStruct((n_pad, hh), jnp.float32),
            jax.ShapeDtypeStruct((n_pad, 1), jnp.float32),
            jax.ShapeDtypeStruct((n_pad, 1), jnp.float32),
        ],
    )(p1, p1, p2, p2, xp, nidxp, W, Wb, b2d, att_src, att_dst)


def _tc_invd_body(dpart_ref, invd_ref):
    s = jnp.sum(dpart_ref[...], axis=0, keepdims=True)
    invd_ref[...] = 1.0 / (s + 1e-16)


def _tc_invd(dpart):
    return pl.pallas_call(
        _tc_invd_body,
        out_shape=jax.ShapeDtypeStruct((1, dpart.shape[1]), jnp.float32),
    )(dpart)


def _sc_edge_stats_body(src_hbm, dst_hbm, stats_hbm,
                        dpart_hbm, ee_hbm,
                        asrc_v, adst_v, den_v, src_v, dst_v, ee_v):
    n_pad = asrc_v.shape[0]
    ept = src_v.shape[0]
    c = lax.axis_index("c")
    s = lax.axis_index("s")
    wid = s * NC + c
    base = wid * ept

    pltpu.sync_copy(stats_hbm.at[pl.ds(0, n_pad)], asrc_v)
    pltpu.sync_copy(stats_hbm.at[pl.ds(n_pad, n_pad)], adst_v)
    pltpu.sync_copy(src_hbm.at[pl.ds(base, ept)], src_v)
    pltpu.sync_copy(dst_hbm.at[pl.ds(base, ept)], dst_v)

    def mbody(i, m):
        return jnp.maximum(m, asrc_v[pl.ds(i * 16, 16)])
    m16 = lax.fori_loop(0, n_pad // 16,
                        mbody, jnp.full((16,), -3e38, jnp.float32))
    gmax = jnp.max(m16)

    def zbody(i, carry):
        den_v[pl.ds(i * 16, 16)] = jnp.zeros((16,), jnp.float32)
        return carry
    lax.fori_loop(0, n_pad // 16, zbody, 0)

    def ebody(g, carry):
        s16 = src_v[pl.ds(g * 16, 16)]
        d16 = dst_v[pl.ds(g * 16, 16)]
        a = plsc.load_gather(asrc_v, [s16])
        bb = plsc.load_gather(adst_v, [d16])
        e0 = a + bb
        e = jnp.where(e0 >= 0, e0, e0 * 0.2)
        c0 = gmax + bb
        cc = jnp.where(c0 >= 0, c0, c0 * 0.2)
        ee = jnp.exp(e - cc)
        ee_v[pl.ds(g * 16, 16)] = ee
        lane = lax.iota(jnp.int32, 16)
        for l in range(16):
            plsc.addupdate_scatter(den_v, [d16], ee, mask=lane == l)
        return carry
    lax.fori_loop(0, ept // 16, ebody, 0)

    pltpu.sync_copy(ee_v, ee_hbm.at[pl.ds(base, ept)])
    pltpu.sync_copy(den_v, dpart_hbm.at[pl.ds(wid * n_pad, n_pad)])


def _sc_edge_stats(src, dst, stats, n_pad, ept):
    e = src.shape[0]
    mesh = plsc.VectorSubcoreMesh(core_axis_name="c", subcore_axis_name="s",
                                  num_cores=NC, num_subcores=NS)
    return pl.kernel(
        _sc_edge_stats_body,
        out_type=[
            jax.ShapeDtypeStruct((NW * n_pad,), jnp.float32),
            jax.ShapeDtypeStruct((e,), jnp.float32),
        ],
        mesh=mesh,
        compiler_params=pltpu.CompilerParams(needs_layout_passes=False),
        scratch_types=[
            pltpu.VMEM((n_pad,), jnp.float32),
            pltpu.VMEM((n_pad,), jnp.float32),
            pltpu.VMEM((n_pad,), jnp.float32),
            pltpu.VMEM((ept,), jnp.int32),
            pltpu.VMEM((ept,), jnp.int32),
            pltpu.VMEM((ept,), jnp.float32),
        ],
    )(src, dst, stats)


def _sc_aggregate_body(src_hbm, dst_hbm, ee_hbm, invd_hbm, h1_hbm, h2_hbm,
                       out1_hbm, out2_hbm,
                       invd_v, src_v, dst_v, ee_v, rows_v,
                       sbuf_v, dbuf_v, sem,
                       out_sh):
    n_pad = invd_v.shape[0]
    ept = src_v.shape[0]
    ch = rows_v.shape[0]
    hh = rows_v.shape[1]
    stripe = n_pad // NS
    c = lax.axis_index("c")
    s = lax.axis_index("s")
    wid = s * NC + c
    base = wid * ept

    pltpu.sync_copy(invd_hbm, invd_v)
    pltpu.sync_copy(src_hbm.at[pl.ds(base, ept)], src_v)
    pltpu.sync_copy(dst_hbm.at[pl.ds(base, ept)], dst_v)
    pltpu.sync_copy(ee_hbm.at[pl.ds(base, ept)], ee_v)

    def wbody(g, carry):
        d16 = dst_v[pl.ds(g * 16, 16)]
        ee_v[pl.ds(g * 16, 16)] = ee_v[pl.ds(g * 16, 16)] * \
            plsc.load_gather(invd_v, [d16])
        return carry
    lax.fori_loop(0, ept // 16, wbody, 0)

    def zbody(i, carry):
        for j in range(hh // 16):
            rows_v[i, pl.ds(j * 16, 16)] = jnp.zeros((16,), jnp.float32)
        return carry
    lax.fori_loop(0, ch, zbody, 0)

    for half, (hsrc, hout) in enumerate(((h1_hbm, out1_hbm),
                                         (h2_hbm, out2_hbm))):
        for r in range(stripe // ch):
            pltpu.sync_copy(rows_v, out_sh.at[pl.ds(s * stripe + r * ch, ch)])
        plsc.subcore_barrier()

        def chunk_body(k, carry):
            cbase = k * ch
            for g in range(ch // 16):
                sbuf_v[pl.ds(g * 16, 16)] = src_v[pl.ds(cbase + g * 16, 16)]
                dbuf_v[pl.ds(g * 16, 16)] = dst_v[pl.ds(cbase + g * 16, 16)]
            pltpu.async_copy(hsrc.at[sbuf_v], rows_v, sem).wait()

            def scale_body(i, carry2):
                ws = plsc.load_gather(
                    ee_v, [jnp.zeros((16,), jnp.int32) + cbase + i])
                for j in range(hh // 16):
                    rows_v[i, pl.ds(j * 16, 16)] = \
                        rows_v[i, pl.ds(j * 16, 16)] * ws
                return carry2
            lax.fori_loop(0, ch, scale_body, 0)
            pltpu.sync_copy(rows_v, out_sh.at[dbuf_v], add=True)
            return carry
        lax.fori_loop(0, ept // ch, chunk_body, 0)

        plsc.subcore_barrier()
        pltpu.sync_copy(out_sh.at[pl.ds(s * stripe, stripe)],
                        hout.at[pl.ds(c * n_pad + s * stripe, stripe)])
        if half == 0:
            def rzbody(i, carry):
                for j in range(hh // 16):
                    rows_v[i, pl.ds(j * 16, 16)] = jnp.zeros((16,),
                                                             jnp.float32)
                return carry
            lax.fori_loop(0, ch, rzbody, 0)


def _sc_aggregate(src, dst, ee, invd, h1, h2, n_pad, ept, ch):
    hh = h1.shape[1]
    mesh = plsc.VectorSubcoreMesh(core_axis_name="c", subcore_axis_name="s",
                                  num_cores=NC, num_subcores=NS)
    return pl.kernel(
        _sc_aggregate_body,
        out_type=[
            jax.ShapeDtypeStruct((NC * n_pad, hh), jnp.float32),
            jax.ShapeDtypeStruct((NC * n_pad, hh), jnp.float32),
        ],
        mesh=mesh,
        compiler_params=pltpu.CompilerParams(needs_layout_passes=False,
                                             use_tc_tiling_on_sc=False),
        scratch_types=[
            pltpu.VMEM((n_pad,), jnp.float32),
            pltpu.VMEM((ept,), jnp.int32),
            pltpu.VMEM((ept,), jnp.int32),
            pltpu.VMEM((ept,), jnp.float32),
            pltpu.VMEM((ch, hh), jnp.float32),
            pltpu.VMEM((ch,), jnp.int32),
            pltpu.VMEM((ch,), jnp.int32),
            pltpu.SemaphoreType.DMA,
            pltpu.VMEM_SHARED((n_pad, hh), jnp.float32),
        ],
    )(src, dst, ee, invd, h1, h2)


def _tc_readout_body(p1_ref, p2_ref, x_ref, tn_ref, w1_ref, b1_ref, w2_ref,
                     b2_ref, w3_ref, b3_ref, y_ref, loss_ref):
    n = x_ref.shape[0]
    half = p1_ref.shape[0] // 2
    state = jnp.concatenate(
        [p1_ref[:n, :] + p1_ref[half:half + n, :],
         p2_ref[:n, :] + p2_ref[half:half + n, :]], axis=1)
    sc = jnp.concatenate([state, x_ref[...]], axis=1)
    h1 = _leaky(sc @ w1_ref[...] + b1_ref[...], 0.01)
    h2 = _leaky(h1 @ w2_ref[...] + b2_ref[...], 0.01)
    y = _leaky(h2 @ w3_ref[...] + b3_ref[...], 0.01)
    y_ref[...] = y
    loss_ref[...] = jnp.mean((y - tn_ref[...]) ** 2).reshape(1, 1)


def _tc_readout(p1, p2, x, tn, w1, b1, w2, b2, w3, b3):
    n = x.shape[0]
    return pl.pallas_call(
        _tc_readout_body,
        out_shape=[
            jax.ShapeDtypeStruct((n, 3), jnp.float32),
            jax.ShapeDtypeStruct((1, 1), jnp.float32),
        ],
    )(p1, p2, x, tn, w1, b1, w2, b2, w3, b3)


def kernel(x, edge_index, target_n, target_g, node_idx, W, Wb, b, att_src,
           att_dst, w1, b1, w2, b2, w3, b3):
    n = x.shape[0]
    h_dim = W.shape[0]
    e_cnt = edge_index.shape[1]
    n_pad = ((n + NW * 16 - 1) // (NW * 16)) * (NW * 16)
    ept = e_cnt // NW
    ch = 80

    src = edge_index[0]
    dst = edge_index[1]
    nidx2d = node_idx.reshape(n, 1)
    b2d = b.reshape(1, h_dim)
    b1_2 = b1.reshape(1, -1)
    b2_2 = b2.reshape(1, -1)
    b3_2 = b3.reshape(1, -1)
    props = 4

    p1 = jnp.zeros((NC * n_pad, h_dim // 2), jnp.float32)
    p2 = jnp.zeros((NC * n_pad, h_dim // 2), jnp.float32)
    xp = jnp.pad(x, ((0, n_pad - n), (0, 0)))
    nidxp = jnp.pad(nidx2d, ((0, n_pad - n), (0, 0)))
    for _ in range(props):
        h1, h2, asrc, adst = _tc_stats(p1, p2, xp, nidxp, W, Wb, b2d,
                                       att_src, att_dst, n_pad)
        stats = jnp.concatenate([asrc[:, 0], adst[:, 0]])
        dpart, ee = _sc_edge_stats(src, dst, stats, n_pad, ept)
        invd = _tc_invd(dpart.reshape(NW, n_pad)).reshape(-1)
        p1, p2 = _sc_aggregate(src, dst, ee, invd, h1, h2, n_pad, ept, ch)

    y, loss = _tc_readout(p1, p2, x, target_n, w1, b1_2, w2, b2_2, w3, b3_2)
    return (y, loss[0, 0])


# speedup vs baseline: 34.1603x; 2.0852x over previous
import functools

import jax
import jax.numpy as jnp
from jax import lax
from jax.experimental import pallas as pl
from jax.experimental.pallas import tpu as pltpu
from jax.experimental.pallas import tpu_sc as plsc

NC = 2
NS = 16
NW = NC * NS


def _leaky(v, slope):
    return jnp.where(v >= 0, v, v * slope)


def _tc_stats_body(pa_ref, pb_ref, x_ref, nidx_ref,
                   W_ref, Wb_ref, b_ref, att_s_ref, att_d_ref,
                   h_ref, asrc_ref, adst_ref):
    state = jnp.concatenate(
        [pa_ref[...] + pb_ref[...], x_ref[...]], axis=1)
    h = state @ jnp.concatenate([W_ref[...], Wb_ref[...]], axis=0) + b_ref[...]
    m = att_s_ref.shape[0]
    n_blk = state.shape[0]
    onehot = (nidx_ref[...] == lax.broadcasted_iota(jnp.int32, (n_blk, m), 1)
              ).astype(jnp.float32)
    a_s = onehot @ att_s_ref[...]
    a_d = onehot @ att_d_ref[...]
    h_ref[...] = h
    asrc_ref[...] = jnp.sum(h * a_s, axis=1, keepdims=True)
    adst_ref[...] = jnp.sum(h * a_d, axis=1, keepdims=True)


def _tc_stats(p, xp, nidxp, W, Wb, b2d, att_src, att_dst, n_pad):
    h_dim = W.shape[0]
    grid = 4
    blk = n_pad // grid
    nb2 = n_pad // blk
    full = lambda shape: pl.BlockSpec(shape, lambda g: (0, 0))
    return pl.pallas_call(
        _tc_stats_body,
        grid=(grid,),
        in_specs=[
            pl.BlockSpec((blk, h_dim), lambda g: (g, 0)),
            pl.BlockSpec((blk, h_dim), lambda g, nb2=nb2: (g + nb2, 0)),
            pl.BlockSpec((blk, 2), lambda g: (g, 0)),
            pl.BlockSpec((blk, 1), lambda g: (g, 0)),
            full((h_dim, h_dim)),
            full((2, h_dim)),
            full((1, h_dim)),
            full((att_src.shape[0], h_dim)),
            full((att_src.shape[0], h_dim)),
        ],
        out_specs=[
            pl.BlockSpec((blk, h_dim), lambda g: (g, 0)),
            pl.BlockSpec((blk, 1), lambda g: (g, 0)),
            pl.BlockSpec((blk, 1), lambda g: (g, 0)),
        ],
        out_shape=[
            jax.ShapeDtypeStruct((n_pad, h_dim), jnp.float32),
            jax.ShapeDtypeStruct((n_pad, 1), jnp.float32),
            jax.ShapeDtypeStruct((n_pad, 1), jnp.float32),
        ],
    )(p, p, xp, nidxp, W, Wb, b2d, att_src, att_dst)


def _tc_invd_body(dpart_ref, invd_ref):
    s = jnp.sum(dpart_ref[...], axis=0, keepdims=True)
    invd_ref[...] = 1.0 / (s + 1e-16)


def _tc_invd(dpart):
    return pl.pallas_call(
        _tc_invd_body,
        out_shape=jax.ShapeDtypeStruct((1, dpart.shape[1]), jnp.float32),
    )(dpart)


def _sc_edge_stats_body(src_hbm, dst_hbm, stats_hbm,
                        dpart_hbm, ee_hbm,
                        asrc_v, adst_v, den_v, src_v, dst_v, ee_v):
    n_pad = asrc_v.shape[0]
    ept = src_v.shape[0]
    c = lax.axis_index("c")
    s = lax.axis_index("s")
    wid = s * NC + c
    base = wid * ept

    pltpu.sync_copy(stats_hbm.at[pl.ds(0, n_pad)], asrc_v)
    pltpu.sync_copy(stats_hbm.at[pl.ds(n_pad, n_pad)], adst_v)
    pltpu.sync_copy(src_hbm.at[pl.ds(base, ept)], src_v)
    pltpu.sync_copy(dst_hbm.at[pl.ds(base, ept)], dst_v)

    def mbody(i, m):
        return jnp.maximum(m, asrc_v[pl.ds(i * 16, 16)])
    m16 = lax.fori_loop(0, n_pad // 16,
                        mbody, jnp.full((16,), -3e38, jnp.float32))
    gmax = jnp.max(m16)

    def zbody(i, carry):
        den_v[pl.ds(i * 16, 16)] = jnp.zeros((16,), jnp.float32)
        return carry
    lax.fori_loop(0, n_pad // 16, zbody, 0)

    def ebody(g, carry):
        s16 = src_v[pl.ds(g * 16, 16)]
        d16 = dst_v[pl.ds(g * 16, 16)]
        a = plsc.load_gather(asrc_v, [s16])
        bb = plsc.load_gather(adst_v, [d16])
        e0 = a + bb
        e = jnp.where(e0 >= 0, e0, e0 * 0.2)
        c0 = gmax + bb
        cc = jnp.where(c0 >= 0, c0, c0 * 0.2)
        ee = jnp.exp(e - cc)
        ee_v[pl.ds(g * 16, 16)] = ee
        lane = lax.iota(jnp.int32, 16)
        for l in range(16):
            plsc.addupdate_scatter(den_v, [d16], ee, mask=lane == l)
        return carry
    lax.fori_loop(0, ept // 16, ebody, 0)

    pltpu.sync_copy(ee_v, ee_hbm.at[pl.ds(base, ept)])
    pltpu.sync_copy(den_v, dpart_hbm.at[pl.ds(wid * n_pad, n_pad)])


def _sc_edge_stats(src, dst, stats, n_pad, ept):
    e = src.shape[0]
    mesh = plsc.VectorSubcoreMesh(core_axis_name="c", subcore_axis_name="s",
                                  num_cores=NC, num_subcores=NS)
    return pl.kernel(
        _sc_edge_stats_body,
        out_type=[
            jax.ShapeDtypeStruct((NW * n_pad,), jnp.float32),
            jax.ShapeDtypeStruct((e,), jnp.float32),
        ],
        mesh=mesh,
        compiler_params=pltpu.CompilerParams(needs_layout_passes=False),
        scratch_types=[
            pltpu.VMEM((n_pad,), jnp.float32),
            pltpu.VMEM((n_pad,), jnp.float32),
            pltpu.VMEM((n_pad,), jnp.float32),
            pltpu.VMEM((ept,), jnp.int32),
            pltpu.VMEM((ept,), jnp.int32),
            pltpu.VMEM((ept,), jnp.float32),
        ],
    )(src, dst, stats)


def _sc_aggregate_body(ept, src_hbm, dst_hbm, ee_hbm, invd_hbm, h_hbm,
                       out_hbm,
                       invd_v, seg_src, seg_dst, seg_ee,
                       rows_a, rows_b, sbuf_a, dbuf_a, sbuf_b, dbuf_b,
                       sem_a, sem_b, out_sh):
    n_pad = invd_v.shape[0]
    seg = seg_src.shape[0]
    ch = rows_a.shape[0]
    hd = rows_a.shape[1]
    nseg = ept // seg
    cps = seg // ch
    stripe = n_pad // NS
    c = lax.axis_index("c")
    s = lax.axis_index("s")
    wid = s * NC + c
    base = wid * ept

    pltpu.sync_copy(invd_hbm, invd_v)

    def zbody(i, carry):
        for j in range(hd // 16):
            rows_a[i, pl.ds(j * 16, 16)] = jnp.zeros((16,), jnp.float32)
        return carry
    lax.fori_loop(0, ch, zbody, 0)
    for r in range(stripe // ch):
        pltpu.sync_copy(rows_a, out_sh.at[pl.ds(s * stripe + r * ch, ch)])
    plsc.subcore_barrier()

    def prep_fire(k, sbuf, dbuf, rows, sem):
        cb = k * ch
        for g in range(ch // 16):
            sbuf[pl.ds(g * 16, 16)] = seg_src[pl.ds(cb + g * 16, 16)]
            dbuf[pl.ds(g * 16, 16)] = seg_dst[pl.ds(cb + g * 16, 16)]
        pltpu.async_copy(h_hbm.at[sbuf], rows, sem)

    def scale_scatter(k, dbuf, rows):
        cb = k * ch

        def sbody(i, carry2):
            e0 = 2 * i
            for e in (e0, e0 + 1):
                ws = plsc.load_gather(
                    seg_ee, [jnp.zeros((16,), jnp.int32) + cb + e])
                for j in range(hd // 16):
                    rows[e, pl.ds(j * 16, 16)] = \
                        rows[e, pl.ds(j * 16, 16)] * ws
            return carry2
        lax.fori_loop(0, ch // 2, sbody, 0)
        pltpu.sync_copy(rows, out_sh.at[dbuf], add=True)

    for si in range(nseg):
        sb = base + si * seg
        pltpu.sync_copy(src_hbm.at[pl.ds(sb, seg)], seg_src)
        pltpu.sync_copy(dst_hbm.at[pl.ds(sb, seg)], seg_dst)
        pltpu.sync_copy(ee_hbm.at[pl.ds(sb, seg)], seg_ee)

        def wbody(g, carry):
            d16 = seg_dst[pl.ds(g * 16, 16)]
            seg_ee[pl.ds(g * 16, 16)] = seg_ee[pl.ds(g * 16, 16)] * \
                plsc.load_gather(invd_v, [d16])
            return carry
        lax.fori_loop(0, seg // 16, wbody, 0)

        prep_fire(0, sbuf_a, dbuf_a, rows_a, sem_a)

        def pipe(i, carry):
            ka = 2 * i
            prep_fire(ka + 1, sbuf_b, dbuf_b, rows_b, sem_b)
            pltpu.make_async_copy(h_hbm.at[sbuf_a], rows_a, sem_a).wait()
            scale_scatter(ka, dbuf_a, rows_a)
            prep_fire(ka + 2, sbuf_a, dbuf_a, rows_a, sem_a)
            pltpu.make_async_copy(h_hbm.at[sbuf_b], rows_b, sem_b).wait()
            scale_scatter(ka + 1, dbuf_b, rows_b)
            return carry
        lax.fori_loop(0, (cps - 1) // 2, pipe, 0)
        pltpu.make_async_copy(h_hbm.at[sbuf_a], rows_a, sem_a).wait()
        scale_scatter(cps - 1, dbuf_a, rows_a)

    plsc.subcore_barrier()
    pltpu.sync_copy(out_sh.at[pl.ds(s * stripe, stripe)],
                    out_hbm.at[pl.ds(c * n_pad + s * stripe, stripe)])


def _sc_aggregate(src, dst, ee, invd, h, n_pad, ept, ch, seg):
    hd = h.shape[1]
    mesh = plsc.VectorSubcoreMesh(core_axis_name="c", subcore_axis_name="s",
                                  num_cores=NC, num_subcores=NS)
    return pl.kernel(
        functools.partial(_sc_aggregate_body, ept),
        out_type=jax.ShapeDtypeStruct((NC * n_pad, hd), jnp.float32),
        mesh=mesh,
        compiler_params=pltpu.CompilerParams(needs_layout_passes=False),
        scratch_types=[
            pltpu.VMEM((n_pad,), jnp.float32),
            pltpu.VMEM((seg,), jnp.int32),
            pltpu.VMEM((seg,), jnp.int32),
            pltpu.VMEM((seg,), jnp.float32),
            pltpu.VMEM((ch, hd), jnp.float32),
            pltpu.VMEM((ch, hd), jnp.float32),
            pltpu.VMEM((ch,), jnp.int32),
            pltpu.VMEM((ch,), jnp.int32),
            pltpu.VMEM((ch,), jnp.int32),
            pltpu.VMEM((ch,), jnp.int32),
            pltpu.SemaphoreType.DMA,
            pltpu.SemaphoreType.DMA,
            pltpu.VMEM_SHARED((n_pad, hd), jnp.float32),
        ],
    )(src, dst, ee, invd, h)


def _tc_readout_body(p_ref, x_ref, tn_ref, w1_ref, b1_ref, w2_ref,
                     b2_ref, w3_ref, b3_ref, y_ref, loss_ref):
    n = x_ref.shape[0]
    half = p_ref.shape[0] // 2
    state = p_ref[:n, :] + p_ref[half:half + n, :]
    sc = jnp.concatenate([state, x_ref[...]], axis=1)
    h1 = _leaky(sc @ w1_ref[...] + b1_ref[...], 0.01)
    h2 = _leaky(h1 @ w2_ref[...] + b2_ref[...], 0.01)
    y = _leaky(h2 @ w3_ref[...] + b3_ref[...], 0.01)
    y_ref[...] = y
    loss_ref[...] = jnp.mean((y - tn_ref[...]) ** 2).reshape(1, 1)


def _tc_readout(p, x, tn, w1, b1, w2, b2, w3, b3):
    n = x.shape[0]
    return pl.pallas_call(
        _tc_readout_body,
        out_shape=[
            jax.ShapeDtypeStruct((n, 3), jnp.float32),
            jax.ShapeDtypeStruct((1, 1), jnp.float32),
        ],
    )(p, x, tn, w1, b1, w2, b2, w3, b3)


def kernel(x, edge_index, target_n, target_g, node_idx, W, Wb, b, att_src,
           att_dst, w1, b1, w2, b2, w3, b3):
    n = x.shape[0]
    h_dim = W.shape[0]
    e_cnt = edge_index.shape[1]
    n_pad = ((n + NW * 16 - 1) // (NW * 16)) * (NW * 16)
    ept = e_cnt // NW
    ch = 80
    seg = 2000

    src = edge_index[0]
    dst = edge_index[1]
    nidx2d = node_idx.reshape(n, 1)
    b2d = b.reshape(1, h_dim)
    b1_2 = b1.reshape(1, -1)
    b2_2 = b2.reshape(1, -1)
    b3_2 = b3.reshape(1, -1)
    props = 4

    p = jnp.zeros((NC * n_pad, h_dim), jnp.float32)
    xp = jnp.pad(x, ((0, n_pad - n), (0, 0)))
    nidxp = jnp.pad(nidx2d, ((0, n_pad - n), (0, 0)))
    for _ in range(props):
        h, asrc, adst = _tc_stats(p, xp, nidxp, W, Wb, b2d,
                                  att_src, att_dst, n_pad)
        stats = jnp.concatenate([asrc[:, 0], adst[:, 0]])
        dpart, ee = _sc_edge_stats(src, dst, stats, n_pad, ept)
        invd = _tc_invd(dpart.reshape(NW, n_pad)).reshape(-1)
        p = _sc_aggregate(src, dst, ee, invd, h, n_pad, ept, ch, seg)

    y, loss = _tc_readout(p, x, target_n, w1, b1_2, w2, b2_2, w3, b3_2)
    return (y, loss[0, 0])


# speedup vs baseline: 36.1698x; 1.0588x over previous
import functools

import jax
import jax.numpy as jnp
from jax import lax
from jax.experimental import pallas as pl
from jax.experimental.pallas import tpu as pltpu
from jax.experimental.pallas import tpu_sc as plsc

NC = 2
NS = 16
NW = NC * NS


def _leaky(v, slope):
    return jnp.where(v >= 0, v, v * slope)


def _tc_stats_body(pa_ref, pb_ref, da_ref, db_ref, x_ref, nidx_ref,
                   W_ref, Wb_ref, b_ref, att_s_ref, att_d_ref,
                   h_ref, asrc_ref, adst_ref):
    inv = 1.0 / (da_ref[...] + db_ref[...] + 1e-16)
    state = jnp.concatenate(
        [(pa_ref[...] + pb_ref[...]) * inv, x_ref[...]], axis=1)
    h = state @ jnp.concatenate([W_ref[...], Wb_ref[...]], axis=0) + b_ref[...]
    m = att_s_ref.shape[0]
    n_blk = state.shape[0]
    onehot = (nidx_ref[...] == lax.broadcasted_iota(jnp.int32, (n_blk, m), 1)
              ).astype(jnp.float32)
    a_s = onehot @ att_s_ref[...]
    a_d = onehot @ att_d_ref[...]
    h_ref[...] = h
    asrc_ref[...] = jnp.sum(h * a_s, axis=1, keepdims=True)
    adst_ref[...] = jnp.sum(h * a_d, axis=1, keepdims=True)


def _tc_stats(p, den2d, xp, nidxp, W, Wb, b2d, att_src, att_dst, n_pad):
    h_dim = W.shape[0]
    grid = 4
    blk = n_pad // grid
    nb2 = n_pad // blk
    full = lambda shape: pl.BlockSpec(shape, lambda g: (0, 0))
    return pl.pallas_call(
        _tc_stats_body,
        grid=(grid,),
        in_specs=[
            pl.BlockSpec((blk, h_dim), lambda g: (g, 0)),
            pl.BlockSpec((blk, h_dim), lambda g, nb2=nb2: (g + nb2, 0)),
            pl.BlockSpec((blk, 1), lambda g: (g, 0)),
            pl.BlockSpec((blk, 1), lambda g, nb2=nb2: (g + nb2, 0)),
            pl.BlockSpec((blk, 2), lambda g: (g, 0)),
            pl.BlockSpec((blk, 1), lambda g: (g, 0)),
            full((h_dim, h_dim)),
            full((2, h_dim)),
            full((1, h_dim)),
            full((att_src.shape[0], h_dim)),
            full((att_src.shape[0], h_dim)),
        ],
        out_specs=[
            pl.BlockSpec((blk, h_dim), lambda g: (g, 0)),
            pl.BlockSpec((blk, 1), lambda g: (g, 0)),
            pl.BlockSpec((blk, 1), lambda g: (g, 0)),
        ],
        out_shape=[
            jax.ShapeDtypeStruct((n_pad, h_dim), jnp.float32),
            jax.ShapeDtypeStruct((n_pad, 1), jnp.float32),
            jax.ShapeDtypeStruct((n_pad, 1), jnp.float32),
        ],
    )(p, p, den2d, den2d, xp, nidxp, W, Wb, b2d, att_src, att_dst)


def _sc_edges_body(ept, src_hbm, dst_hbm, asrc_hbm, adst_hbm, h_hbm,
                   out_hbm, den_hbm,
                   asrc_v, adst_v, seg_src, seg_dst, ee_v,
                   rows_a, rows_b, sbuf_a, dbuf_a, sbuf_b, dbuf_b,
                   gsem_a, gsem_b, ssem_a, ssem_b, dsem_a, dsem_b,
                   out_sh, den_sh):
    n_pad = asrc_v.shape[0]
    seg = seg_src.shape[0]
    ch = rows_a.shape[0]
    hd = rows_a.shape[1]
    nseg = ept // seg
    cps = seg // ch
    stripe = n_pad // NS
    c = lax.axis_index("c")
    s = lax.axis_index("s")
    wid = s * NC + c
    base = wid * ept

    pltpu.sync_copy(asrc_hbm, asrc_v)
    pltpu.sync_copy(adst_hbm, adst_v)

    def mbody(i, m):
        return jnp.maximum(m, asrc_v[pl.ds(i * 16, 16)])
    m16 = lax.fori_loop(0, n_pad // 16,
                        mbody, jnp.full((16,), -3e38, jnp.float32))
    gmax = jnp.max(m16)

    def zbody(i, carry):
        for j in range(hd // 16):
            rows_a[i, pl.ds(j * 16, 16)] = jnp.zeros((16,), jnp.float32)
        return carry
    lax.fori_loop(0, ch, zbody, 0)
    def zbody2(i, carry):
        ee_v[pl.ds(i * 16, 16)] = jnp.zeros((16,), jnp.float32)
        return carry
    lax.fori_loop(0, stripe // 16, zbody2, 0)
    for r in range(stripe // ch):
        pltpu.sync_copy(rows_a, out_sh.at[pl.ds(s * stripe + r * ch, ch)])
    pltpu.sync_copy(ee_v.at[pl.ds(0, stripe)],
                    den_sh.at[pl.ds(s * stripe, stripe)])
    plsc.subcore_barrier()

    def prep_fire(k, sbuf, dbuf, rows, gsem):
        cb = k * ch
        for g in range(ch // 16):
            sbuf[pl.ds(g * 16, 16)] = seg_src[pl.ds(cb + g * 16, 16)]
            dbuf[pl.ds(g * 16, 16)] = seg_dst[pl.ds(cb + g * 16, 16)]
        pltpu.async_copy(h_hbm.at[sbuf], rows, gsem)

    def ee_fire(k, sbuf, dbuf, dsem):
        cb = k * ch
        for g in range(ch // 16):
            s16 = sbuf[pl.ds(g * 16, 16)]
            d16 = dbuf[pl.ds(g * 16, 16)]
            a = plsc.load_gather(asrc_v, [s16])
            bb = plsc.load_gather(adst_v, [d16])
            e0 = a + bb
            e = jnp.where(e0 >= 0, e0, e0 * 0.2)
            c0 = gmax + bb
            cc = jnp.where(c0 >= 0, c0, c0 * 0.2)
            ee_v[pl.ds(cb + g * 16, 16)] = jnp.exp(e - cc)
        pltpu.async_copy(ee_v.at[pl.ds(cb, ch)], den_sh.at[dbuf], dsem,
                         add=True)

    def wait_g(sbuf, rows, gsem):
        pltpu.make_async_copy(h_hbm.at[sbuf], rows, gsem).wait()

    def fire_s(rows, dbuf, ssem):
        pltpu.async_copy(rows, out_sh.at[dbuf], ssem, add=True)

    def wait_s(rows, dbuf, ssem):
        pltpu.make_async_copy(rows, out_sh.at[dbuf], ssem).wait()

    def wait_d(dbuf, dsem):
        pltpu.make_async_copy(ee_v.at[pl.ds(0, ch)], den_sh.at[dbuf],
                              dsem).wait()

    def scale(k, rows):
        cb = k * ch

        def sbody(i, carry2):
            e0 = 4 * i
            for e in (e0, e0 + 1, e0 + 2, e0 + 3):
                ws = plsc.load_gather(
                    ee_v, [jnp.zeros((16,), jnp.int32) + cb + e])
                for j in range(hd // 16):
                    rows[e, pl.ds(j * 16, 16)] = \
                        rows[e, pl.ds(j * 16, 16)] * ws
            return carry2
        lax.fori_loop(0, ch // 4, sbody, 0)

    for si in range(nseg):
        sb = base + si * seg
        pltpu.sync_copy(src_hbm.at[pl.ds(sb, seg)], seg_src)
        pltpu.sync_copy(dst_hbm.at[pl.ds(sb, seg)], seg_dst)

        prep_fire(0, sbuf_a, dbuf_a, rows_a, gsem_a)
        ee_fire(0, sbuf_a, dbuf_a, dsem_a)
        prep_fire(1, sbuf_b, dbuf_b, rows_b, gsem_b)
        ee_fire(1, sbuf_b, dbuf_b, dsem_b)
        wait_g(sbuf_a, rows_a, gsem_a)
        scale(0, rows_a)
        fire_s(rows_a, dbuf_a, ssem_a)
        wait_g(sbuf_b, rows_b, gsem_b)
        scale(1, rows_b)
        fire_s(rows_b, dbuf_b, ssem_b)
        wait_s(rows_a, dbuf_a, ssem_a)
        wait_d(dbuf_a, dsem_a)
        prep_fire(2, sbuf_a, dbuf_a, rows_a, gsem_a)
        ee_fire(2, sbuf_a, dbuf_a, dsem_a)

        def pipe(i, carry):
            k = 2 * i + 2
            wait_s(rows_b, dbuf_b, ssem_b)
            wait_d(dbuf_b, dsem_b)
            prep_fire(k + 1, sbuf_b, dbuf_b, rows_b, gsem_b)
            ee_fire(k + 1, sbuf_b, dbuf_b, dsem_b)
            wait_g(sbuf_a, rows_a, gsem_a)
            scale(k, rows_a)
            fire_s(rows_a, dbuf_a, ssem_a)
            wait_g(sbuf_b, rows_b, gsem_b)
            scale(k + 1, rows_b)
            fire_s(rows_b, dbuf_b, ssem_b)
            wait_s(rows_a, dbuf_a, ssem_a)
            wait_d(dbuf_a, dsem_a)
            prep_fire(k + 2, sbuf_a, dbuf_a, rows_a, gsem_a)
            ee_fire(k + 2, sbuf_a, dbuf_a, dsem_a)
            return carry
        lax.fori_loop(0, (cps - 3) // 2, pipe, 0)
        wait_g(sbuf_a, rows_a, gsem_a)
        scale(cps - 1, rows_a)
        fire_s(rows_a, dbuf_a, ssem_a)
        wait_s(rows_b, dbuf_b, ssem_b)
        wait_d(dbuf_b, dsem_b)
        wait_s(rows_a, dbuf_a, ssem_a)
        wait_d(dbuf_a, dsem_a)

    plsc.subcore_barrier()
    pltpu.sync_copy(out_sh.at[pl.ds(s * stripe, stripe)],
                    out_hbm.at[pl.ds(c * n_pad + s * stripe, stripe)])
    pltpu.sync_copy(den_sh.at[pl.ds(s * stripe, stripe)],
                    den_hbm.at[pl.ds(c * n_pad + s * stripe, stripe)])


def _sc_edges(src, dst, asrc, adst, h, n_pad, ept, ch, seg):
    hd = h.shape[1]
    mesh = plsc.VectorSubcoreMesh(core_axis_name="c", subcore_axis_name="s",
                                  num_cores=NC, num_subcores=NS)
    return pl.kernel(
        functools.partial(_sc_edges_body, ept),
        out_type=[
            jax.ShapeDtypeStruct((NC * n_pad, hd), jnp.float32),
            jax.ShapeDtypeStruct((NC * n_pad,), jnp.float32),
        ],
        mesh=mesh,
        compiler_params=pltpu.CompilerParams(needs_layout_passes=False),
        scratch_types=[
            pltpu.VMEM((n_pad,), jnp.float32),
            pltpu.VMEM((n_pad,), jnp.float32),
            pltpu.VMEM((seg,), jnp.int32),
            pltpu.VMEM((seg,), jnp.int32),
            pltpu.VMEM((seg,), jnp.float32),
            pltpu.VMEM((ch, hd), jnp.float32),
            pltpu.VMEM((ch, hd), jnp.float32),
            pltpu.VMEM((ch,), jnp.int32),
            pltpu.VMEM((ch,), jnp.int32),
            pltpu.VMEM((ch,), jnp.int32),
            pltpu.VMEM((ch,), jnp.int32),
            pltpu.SemaphoreType.DMA,
            pltpu.SemaphoreType.DMA,
            pltpu.SemaphoreType.DMA,
            pltpu.SemaphoreType.DMA,
            pltpu.SemaphoreType.DMA,
            pltpu.SemaphoreType.DMA,
            pltpu.VMEM_SHARED((n_pad, hd), jnp.float32),
            pltpu.VMEM_SHARED((n_pad,), jnp.float32),
        ],
    )(src, dst, asrc, adst, h)


def _tc_readout_body(p_ref, den_ref, x_ref, tn_ref, w1_ref, b1_ref, w2_ref,
                     b2_ref, w3_ref, b3_ref, y_ref, loss_ref):
    n = x_ref.shape[0]
    half = p_ref.shape[0] // 2
    inv = 1.0 / (den_ref[:n, :] + den_ref[half:half + n, :] + 1e-16)
    state = (p_ref[:n, :] + p_ref[half:half + n, :]) * inv
    sc = jnp.concatenate([state, x_ref[...]], axis=1)
    h1 = _leaky(sc @ w1_ref[...] + b1_ref[...], 0.01)
    h2 = _leaky(h1 @ w2_ref[...] + b2_ref[...], 0.01)
    y = _leaky(h2 @ w3_ref[...] + b3_ref[...], 0.01)
    y_ref[...] = y
    loss_ref[...] = jnp.mean((y - tn_ref[...]) ** 2).reshape(1, 1)


def _tc_readout(p, den2d, x, tn, w1, b1, w2, b2, w3, b3):
    n = x.shape[0]
    return pl.pallas_call(
        _tc_readout_body,
        out_shape=[
            jax.ShapeDtypeStruct((n, 3), jnp.float32),
            jax.ShapeDtypeStruct((1, 1), jnp.float32),
        ],
    )(p, den2d, x, tn, w1, b1, w2, b2, w3, b3)


def kernel(x, edge_index, target_n, target_g, node_idx, W, Wb, b, att_src,
           att_dst, w1, b1, w2, b2, w3, b3):
    n = x.shape[0]
    h_dim = W.shape[0]
    e_cnt = edge_index.shape[1]
    n_pad = ((n + NW * 16 - 1) // (NW * 16)) * (NW * 16)
    ept = e_cnt // NW
    ch = 80
    seg = 2000

    src = edge_index[0]
    dst = edge_index[1]
    nidx2d = node_idx.reshape(n, 1)
    b2d = b.reshape(1, h_dim)
    b1_2 = b1.reshape(1, -1)
    b2_2 = b2.reshape(1, -1)
    b3_2 = b3.reshape(1, -1)
    props = 4

    p = jnp.zeros((NC * n_pad, h_dim), jnp.float32)
    den = jnp.zeros((NC * n_pad, 1), jnp.float32)
    xp = jnp.pad(x, ((0, n_pad - n), (0, 0)))
    nidxp = jnp.pad(nidx2d, ((0, n_pad - n), (0, 0)))
    for _ in range(props):
        h, asrc, adst = _tc_stats(p, den, xp, nidxp, W, Wb, b2d,
                                  att_src, att_dst, n_pad)
        p, den1 = _sc_edges(src, dst, asrc.reshape(-1), adst.reshape(-1), h,
                            n_pad, ept, ch, seg)
        den = den1.reshape(-1, 1)

    y, loss = _tc_readout(p, den, x, target_n, w1, b1_2, w2, b2_2, w3, b3_2)
    return (y, loss[0, 0])
